# Initial kernel scaffold; baseline (speedup 1.0000x reference)
import jax
import jax.numpy as jnp
from jax.experimental import pallas as pl


def kernel(x, edge_index, batch, c1_Wa, c1_ba, c1_Wb, c1_bb, bn1_g, bn1_b, c2_Wa, c2_ba, c2_Wb, c2_bb, bn2_g, bn2_b, c3_Wa, c3_ba, c3_Wb, c3_bb, bn3_g, bn3_b, fc1_W, fc1_b, fc2_W, fc2_b):
    raise NotImplementedError("write your pallas kernel here")


# speedup vs baseline: 8.8863x; 8.8863x over previous
import functools

import jax
import jax.numpy as jnp
import numpy as np
from jax import lax
from jax.experimental import pallas as pl
from jax.experimental.pallas import tpu as pltpu
from jax.experimental.pallas import tpu_sc as plsc

N = 100000
E = 1600000
G = 1000
GP = 1024
D = 32
H = 16
NP = 102400
EP = 1638400
ER = EP // 128
NPR = NP // 128
NC = 2
NS = 16
R = 1024
BN_SCALE = float(1.0 / (1.0 + 1e-5) ** 0.5)

ETR = ER // NS
SUP = 8
NSUP = ETR // SUP

_PAD_IDX = np.asarray(N + np.arange(EP - E) % 2048, dtype=np.int32)

_sc_mesh = functools.partial(
    plsc.VectorSubcoreMesh,
    core_axis_name="c", subcore_axis_name="s", num_cores=NC, num_subcores=NS,
)
_SC_PARAMS = pltpu.CompilerParams(use_tc_tiling_on_sc=False)


def _agg_body(y_hbm, src_hbm, dst_hbm, out_hbm, acc, sidx, didx, rows, zbuf,
              gsem):
  c = lax.axis_index("c")
  s = lax.axis_index("s")

  def zfill(i, carry):
    zbuf[i] = jnp.zeros((H,), jnp.float32)
    return carry
  lax.fori_loop(0, 400, zfill, 0)
  zrow0 = s * (NP // NS)
  for k in range(16):
    pltpu.sync_copy(zbuf, acc.at[pl.ds(zrow0 + k * 400, 400)])
  plsc.subcore_barrier()

  coff = c * NP
  r0 = s * ETR

  def stage(u, carry):
    jrow = r0 + u * SUP
    pltpu.sync_copy(src_hbm.at[pl.ds(jrow, SUP)], sidx)
    pltpu.sync_copy(dst_hbm.at[pl.ds(jrow, SUP)], didx)
    for r in range(SUP):
      for k in range(8):
        sidx[r, pl.ds(k * 16, 16)] = sidx[r, pl.ds(k * 16, 16)] + coff
    descs = [
        pltpu.async_copy(y_hbm.at[sidx.at[r]], rows.at[r], gsem)
        for r in range(SUP)
    ]
    for d in descs:
      d.wait()
    for r in range(SUP):
      pltpu.sync_copy(rows.at[r], acc.at[didx.at[r]], add=True)
    return carry

  lax.fori_loop(0, NSUP, stage, 0)
  plsc.subcore_barrier()
  wrow0 = s * (NP // NS)
  pltpu.sync_copy(acc.at[pl.ds(wrow0, NP // NS)],
                  out_hbm.at[pl.ds(coff + wrow0, NP // NS)])


@functools.cache
def _agg_call():
  return pl.kernel(
      _agg_body,
      out_type=jax.ShapeDtypeStruct((NC * NP, H), jnp.float32),
      mesh=_sc_mesh(),
      scratch_types=[
          pltpu.VMEM_SHARED((NP, H), jnp.float32),
          pltpu.VMEM((SUP, 128), jnp.int32),
          pltpu.VMEM((SUP, 128), jnp.int32),
          pltpu.VMEM((SUP, 128, H), jnp.float32),
          pltpu.VMEM((400, H), jnp.float32),
          pltpu.SemaphoreType.DMA,
      ],
      compiler_params=_SC_PARAMS,
  )


def _pool_body(h_hbm, b_hbm, out_hbm, acc, bidx, vrows, zbuf, gsem):
  c = lax.axis_index("c")
  s = lax.axis_index("s")

  def zfill(i, carry):
    zbuf[i] = jnp.zeros((H,), jnp.float32)
    return carry
  lax.fori_loop(0, 64, zfill, 0)
  pltpu.sync_copy(zbuf, acc.at[pl.ds(s * 64, 64)])
  plsc.subcore_barrier()

  def stage(u, carry):
    g = s + u * NS
    pltpu.sync_copy(b_hbm.at[pl.ds(g * 8, 8)], bidx)
    pltpu.sync_copy(h_hbm.at[pl.ds(c * NP + g * 1024, 1024)], vrows)
    for r in range(8):
      pltpu.sync_copy(vrows.at[pl.ds(r * 128, 128)], acc.at[bidx.at[r]],
                      add=True)
    return carry
  lax.fori_loop(0, 6 + (s < 4).astype(jnp.int32), stage, 0)

  plsc.subcore_barrier()
  pltpu.sync_copy(acc.at[pl.ds(s * 64, 64)],
                  out_hbm.at[pl.ds(c * GP + s * 64, 64)])


@functools.cache
def _pool_call():
  return pl.kernel(
      _pool_body,
      out_type=jax.ShapeDtypeStruct((NC * GP, H), jnp.float32),
      mesh=_sc_mesh(),
      scratch_types=[
          pltpu.VMEM_SHARED((GP, H), jnp.float32),
          pltpu.VMEM((8, 128), jnp.int32),
          pltpu.VMEM((1024, H), jnp.float32),
          pltpu.VMEM((64, H), jnp.float32),
          pltpu.SemaphoreType.DMA,
      ],
      compiler_params=_SC_PARAMS,
  )


def _embed_body(x_ref, w_ref, o_ref):
  x = x_ref[...]
  w = w_ref[...]
  y = (x[:, 0:1] * w[0:1, :] + x[:, 1:2] * w[1:2, :] + x[:, 2:3] * w[2:3, :])
  o_ref[0] = y[:, :H]
  o_ref[1] = y[:, H:]


def _embed(x, wa):
  return pl.pallas_call(
      _embed_body,
      grid=(NP // R,),
      in_specs=[
          pl.BlockSpec((R, 3), lambda i: (i, 0)),
          pl.BlockSpec((3, D), lambda i: (0, 0)),
      ],
      out_specs=pl.BlockSpec((2, R, H), lambda i: (0, i, 0)),
      out_shape=jax.ShapeDtypeStruct((2, NP, H), jnp.float32),
  )(x, wa)


def _mlp(y_ref, a_ref, ba_ref, wb_ref, bb_ref, g_ref, b_ref):
  h = jnp.concatenate([y_ref[0] + a_ref[0], y_ref[1] + a_ref[1]], axis=1)
  h = jnp.maximum(h + ba_ref[...], 0.0)
  u = jnp.maximum(jnp.dot(h, wb_ref[...],
                          preferred_element_type=jnp.float32) + bb_ref[...],
                  0.0)
  return u * (g_ref[...] * BN_SCALE) + b_ref[...]


def _layer_body(y_ref, a_ref, ba_ref, wb_ref, bb_ref, g_ref, b_ref, wn_ref,
                o_ref):
  v = _mlp(y_ref, a_ref, ba_ref, wb_ref, bb_ref, g_ref, b_ref)
  z = jnp.dot(v, wn_ref[...], preferred_element_type=jnp.float32)
  o_ref[0] = z[:, :H]
  o_ref[1] = z[:, H:]


def _final_body(y_ref, a_ref, ba_ref, wb_ref, bb_ref, g_ref, b_ref, o_ref):
  i = pl.program_id(0)
  v = _mlp(y_ref, a_ref, ba_ref, wb_ref, bb_ref, g_ref, b_ref)
  row = i * R + lax.broadcasted_iota(jnp.int32, (R, 1), 0)
  v = jnp.where(row < N, v, 0.0)
  o_ref[0] = v[:, :H]
  o_ref[1] = v[:, H:]


def _split_specs():
  return [
      pl.BlockSpec((2, R, H), lambda i: (0, i, 0)),
      pl.BlockSpec((2, R, H), lambda i: (0, i, 0)),
      pl.BlockSpec((1, D), lambda i: (0, 0)),
      pl.BlockSpec((D, D), lambda i: (0, 0)),
      pl.BlockSpec((1, D), lambda i: (0, 0)),
      pl.BlockSpec((1, D), lambda i: (0, 0)),
      pl.BlockSpec((1, D), lambda i: (0, 0)),
  ]


def _layer(y, a, ba, wb, bb, g, b, wn):
  return pl.pallas_call(
      _layer_body,
      grid=(NP // R,),
      in_specs=_split_specs() + [pl.BlockSpec((D, D), lambda i: (0, 0))],
      out_specs=pl.BlockSpec((2, R, H), lambda i: (0, i, 0)),
      out_shape=jax.ShapeDtypeStruct((2, NP, H), jnp.float32),
  )(y, a, ba.reshape(1, D), wb, bb.reshape(1, D), g.reshape(1, D),
    b.reshape(1, D), wn)


def _final(y, a, ba, wb, bb, g, b):
  return pl.pallas_call(
      _final_body,
      grid=(NP // R,),
      in_specs=_split_specs(),
      out_specs=pl.BlockSpec((2, R, H), lambda i: (0, i, 0)),
      out_shape=jax.ShapeDtypeStruct((2, NP, H), jnp.float32),
  )(y, a, ba.reshape(1, D), wb, bb.reshape(1, D), g.reshape(1, D),
    b.reshape(1, D))


def _head_body(p_ref, w1_ref, b1_ref, w2_ref, b2_ref, o_ref):
  p = jnp.concatenate([p_ref[0], p_ref[1]], axis=1)
  h = jnp.maximum(jnp.dot(p, w1_ref[...],
                          preferred_element_type=jnp.float32) + b1_ref[...],
                  0.0)
  o = jnp.sum(h * w2_ref[...], axis=1, keepdims=True) + b2_ref[...]
  o_ref[...] = jnp.tanh(o[:G])


def _head(pooled, w1, b1, w2, b2):
  return pl.pallas_call(
      _head_body,
      out_shape=jax.ShapeDtypeStruct((G, 1), jnp.float32),
  )(pooled, w1, b1.reshape(1, D), w2.reshape(1, D), b2.reshape(1, 1))


def kernel(x, edge_index, batch,
           c1_Wa, c1_ba, c1_Wb, c1_bb, bn1_g, bn1_b,
           c2_Wa, c2_ba, c2_Wb, c2_bb, bn2_g, bn2_b,
           c3_Wa, c3_ba, c3_Wb, c3_bb, bn3_g, bn3_b,
           fc1_W, fc1_b, fc2_W, fc2_b):
  pad_idx = jnp.asarray(_PAD_IDX)
  src = jnp.concatenate(
      [edge_index[0].astype(jnp.int32), pad_idx]).reshape(ER, 128)
  dst = jnp.concatenate(
      [edge_index[1].astype(jnp.int32), pad_idx]).reshape(ER, 128)
  bpad = jnp.concatenate(
      [batch.astype(jnp.int32), jnp.full((NP - N,), G - 1, jnp.int32)]
  ).reshape(NPR, 128)
  xp = jnp.pad(x, ((0, NP - N), (0, 0)))

  agg = _agg_call()
  y = _embed(xp, c1_Wa)
  a = agg(y.reshape(NC * NP, H), src, dst).reshape(2, NP, H)
  y = _layer(y, a, c1_ba, c1_Wb, c1_bb, bn1_g, bn1_b, c2_Wa)
  a = agg(y.reshape(NC * NP, H), src, dst).reshape(2, NP, H)
  y = _layer(y, a, c2_ba, c2_Wb, c2_bb, bn2_g, bn2_b, c3_Wa)
  a = agg(y.reshape(NC * NP, H), src, dst).reshape(2, NP, H)
  h3 = _final(y, a, c3_ba, c3_Wb, c3_bb, bn3_g, bn3_b)
  pooled = _pool_call()(h3.reshape(NC * NP, H), bpad).reshape(2, GP, H)
  return _head(pooled, fc1_W, fc1_b, fc2_W, fc2_b)


# speedup vs baseline: 12.4527x; 1.4013x over previous
import functools

import jax
import jax.numpy as jnp
import numpy as np
from jax import lax
from jax.experimental import pallas as pl
from jax.experimental.pallas import tpu as pltpu
from jax.experimental.pallas import tpu_sc as plsc

N = 100000
E = 1600000
G = 1000
GP = 1024
D = 32
H = 16
NP = 102400
EP = 1638400
ER = EP // 128
NPR = NP // 128
NC = 2
NS = 16
R = 1024
BN_SCALE = float(1.0 / (1.0 + 1e-5) ** 0.5)

ETR = ER // NS
SUP = 8
NSUP = ETR // SUP

_PAD_IDX = np.asarray(N + np.arange(EP - E) % 2048, dtype=np.int32)

_sc_mesh = functools.partial(
    plsc.VectorSubcoreMesh,
    core_axis_name="c", subcore_axis_name="s", num_cores=NC, num_subcores=NS,
)
_SC_PARAMS = pltpu.CompilerParams(use_tc_tiling_on_sc=False)


def _agg_body(y_hbm, src_hbm, dst_hbm, out_hbm, acc, sidx, didx, rows, zbuf,
              gsem):
  c = lax.axis_index("c")
  s = lax.axis_index("s")

  def zfill(i, carry):
    zbuf[i] = jnp.zeros((H,), jnp.float32)
    return carry
  lax.fori_loop(0, 400, zfill, 0)
  zrow0 = s * (NP // NS)
  for k in range(16):
    pltpu.sync_copy(zbuf, acc.at[pl.ds(zrow0 + k * 400, 400)])
  plsc.subcore_barrier()

  coff = c * NP
  r0 = s * ETR

  def stage(u, carry):
    jrow = r0 + u * SUP
    pltpu.sync_copy(src_hbm.at[pl.ds(jrow, SUP)], sidx)
    pltpu.sync_copy(dst_hbm.at[pl.ds(jrow, SUP)], didx)
    for r in range(SUP):
      for k in range(8):
        sidx[r, pl.ds(k * 16, 16)] = sidx[r, pl.ds(k * 16, 16)] + coff
    descs = [
        pltpu.async_copy(y_hbm.at[sidx.at[r]], rows.at[r], gsem)
        for r in range(SUP)
    ]
    for d in descs:
      d.wait()
    for r in range(SUP):
      pltpu.sync_copy(rows.at[r], acc.at[didx.at[r]], add=True)
    return carry

  lax.fori_loop(0, NSUP, stage, 0)
  plsc.subcore_barrier()
  wrow0 = s * (NP // NS)
  pltpu.sync_copy(acc.at[pl.ds(wrow0, NP // NS)],
                  out_hbm.at[pl.ds(coff + wrow0, NP // NS)])


@functools.cache
def _agg_call():
  return pl.kernel(
      _agg_body,
      out_type=jax.ShapeDtypeStruct((NC * NP, H), jnp.float32),
      mesh=_sc_mesh(),
      scratch_types=[
          pltpu.VMEM_SHARED((NP, H), jnp.float32),
          pltpu.VMEM((SUP, 128), jnp.int32),
          pltpu.VMEM((SUP, 128), jnp.int32),
          pltpu.VMEM((SUP, 128, H), jnp.float32),
          pltpu.VMEM((400, H), jnp.float32),
          pltpu.SemaphoreType.DMA,
      ],
      compiler_params=_SC_PARAMS,
  )


def _pool_body(h_hbm, b_hbm, out_hbm, acc, bidx, vrows, zbuf, gsem):
  c = lax.axis_index("c")
  s = lax.axis_index("s")

  def zfill(i, carry):
    zbuf[i] = jnp.zeros((H,), jnp.float32)
    return carry
  lax.fori_loop(0, 64, zfill, 0)
  pltpu.sync_copy(zbuf, acc.at[pl.ds(s * 64, 64)])
  plsc.subcore_barrier()

  def stage(u, carry):
    g = s + u * NS
    pltpu.sync_copy(b_hbm.at[pl.ds(g * 8, 8)], bidx)
    pltpu.sync_copy(h_hbm.at[pl.ds(c * NP + g * 1024, 1024)], vrows)
    for r in range(8):
      pltpu.sync_copy(vrows.at[pl.ds(r * 128, 128)], acc.at[bidx.at[r]],
                      add=True)
    return carry
  lax.fori_loop(0, 6 + (s < 4).astype(jnp.int32), stage, 0)

  plsc.subcore_barrier()
  pltpu.sync_copy(acc.at[pl.ds(s * 64, 64)],
                  out_hbm.at[pl.ds(c * GP + s * 64, 64)])


@functools.cache
def _pool_call():
  return pl.kernel(
      _pool_body,
      out_type=jax.ShapeDtypeStruct((NC * GP, H), jnp.float32),
      mesh=_sc_mesh(),
      scratch_types=[
          pltpu.VMEM_SHARED((GP, H), jnp.float32),
          pltpu.VMEM((8, 128), jnp.int32),
          pltpu.VMEM((1024, H), jnp.float32),
          pltpu.VMEM((64, H), jnp.float32),
          pltpu.SemaphoreType.DMA,
      ],
      compiler_params=_SC_PARAMS,
  )


PR = NP // 8
RB = 1280


def _embed_body(x_ref, w_ref, o_ref):
  w = w_ref[...]
  z0, z1 = [], []
  for j in range(8):
    xj = x_ref[:, 16 * j:16 * j + 3]
    y = (xj[:, 0:1] * w[0:1, :] + xj[:, 1:2] * w[1:2, :]
         + xj[:, 2:3] * w[2:3, :])
    z0.append(y[:, :H])
    z1.append(y[:, H:])
  o_ref[0] = jnp.concatenate(z0, axis=1)
  o_ref[1] = jnp.concatenate(z1, axis=1)


def _embed(xpk, wa):
  return pl.pallas_call(
      _embed_body,
      grid=(PR // RB,),
      in_specs=[
          pl.BlockSpec((RB, 128), lambda i: (i, 0)),
          pl.BlockSpec((3, D), lambda i: (0, 0)),
      ],
      out_specs=pl.BlockSpec((2, RB, 128), lambda i: (0, i, 0)),
      out_shape=jax.ShapeDtypeStruct((2, PR, 128), jnp.float32),
  )(xpk, wa)


def _mlp_slice(y_ref, a_ref, j, ba, wb, bb, g2, b2):
  lo, hi = 16 * j, 16 * j + 16
  h = jnp.concatenate(
      [y_ref[0][:, lo:hi] + a_ref[0][:, lo:hi],
       y_ref[1][:, lo:hi] + a_ref[1][:, lo:hi]],
      axis=1)
  h = jnp.maximum(h + ba, 0.0)
  u = jnp.maximum(
      jnp.dot(h, wb, preferred_element_type=jnp.float32) + bb, 0.0)
  return u * g2 + b2


def _layer_body(y_ref, a_ref, ba_ref, wb_ref, bb_ref, g_ref, b_ref, wn_ref,
                o_ref):
  ba = ba_ref[...]
  wb = wb_ref[...]
  bb = bb_ref[...]
  g2 = g_ref[...] * BN_SCALE
  b2 = b_ref[...]
  wn = wn_ref[...]
  z0, z1 = [], []
  for j in range(8):
    v = _mlp_slice(y_ref, a_ref, j, ba, wb, bb, g2, b2)
    z = jnp.dot(v, wn, preferred_element_type=jnp.float32)
    z0.append(z[:, :H])
    z1.append(z[:, H:])
  o_ref[0] = jnp.concatenate(z0, axis=1)
  o_ref[1] = jnp.concatenate(z1, axis=1)


def _final_body(y_ref, a_ref, ba_ref, wb_ref, bb_ref, g_ref, b_ref, o_ref):
  i = pl.program_id(0)
  ba = ba_ref[...]
  wb = wb_ref[...]
  bb = bb_ref[...]
  g2 = g_ref[...] * BN_SCALE
  b2 = b_ref[...]
  row = i * RB + lax.broadcasted_iota(jnp.int32, (RB, 1), 0)
  z0, z1 = [], []
  for j in range(8):
    v = _mlp_slice(y_ref, a_ref, j, ba, wb, bb, g2, b2)
    v = jnp.where(row * 8 + j < N, v, 0.0)
    z0.append(v[:, :H])
    z1.append(v[:, H:])
  o_ref[0] = jnp.concatenate(z0, axis=1)
  o_ref[1] = jnp.concatenate(z1, axis=1)


def _split_specs():
  return [
      pl.BlockSpec((2, RB, 128), lambda i: (0, i, 0)),
      pl.BlockSpec((2, RB, 128), lambda i: (0, i, 0)),
      pl.BlockSpec((1, D), lambda i: (0, 0)),
      pl.BlockSpec((D, D), lambda i: (0, 0)),
      pl.BlockSpec((1, D), lambda i: (0, 0)),
      pl.BlockSpec((1, D), lambda i: (0, 0)),
      pl.BlockSpec((1, D), lambda i: (0, 0)),
  ]


def _layer(y, a, ba, wb, bb, g, b, wn):
  return pl.pallas_call(
      _layer_body,
      grid=(PR // RB,),
      in_specs=_split_specs() + [pl.BlockSpec((D, D), lambda i: (0, 0))],
      out_specs=pl.BlockSpec((2, RB, 128), lambda i: (0, i, 0)),
      out_shape=jax.ShapeDtypeStruct((2, PR, 128), jnp.float32),
  )(y, a, ba.reshape(1, D), wb, bb.reshape(1, D), g.reshape(1, D),
    b.reshape(1, D), wn)


def _final(y, a, ba, wb, bb, g, b):
  return pl.pallas_call(
      _final_body,
      grid=(PR // RB,),
      in_specs=_split_specs(),
      out_specs=pl.BlockSpec((2, RB, 128), lambda i: (0, i, 0)),
      out_shape=jax.ShapeDtypeStruct((2, PR, 128), jnp.float32),
  )(y, a, ba.reshape(1, D), wb, bb.reshape(1, D), g.reshape(1, D),
    b.reshape(1, D))


def _head_body(p_ref, w1_ref, b1_ref, w2_ref, b2_ref, o_ref):
  w1 = w1_ref[...]
  b1 = b1_ref[...]
  w2 = w2_ref[...]
  cols = []
  for j in range(8):
    lo, hi = 16 * j, 16 * j + 16
    p = jnp.concatenate([p_ref[0][:, lo:hi], p_ref[1][:, lo:hi]], axis=1)
    h = jnp.maximum(
        jnp.dot(p, w1, preferred_element_type=jnp.float32) + b1, 0.0)
    cols.append(jnp.sum(h * w2, axis=1, keepdims=True))
  o = jnp.concatenate(cols, axis=1) + b2_ref[...]
  o_ref[...] = jnp.tanh(o)


def _head(pooled_pk, w1, b1, w2, b2):
  return pl.pallas_call(
      _head_body,
      out_shape=jax.ShapeDtypeStruct((GP // 8, 8), jnp.float32),
  )(pooled_pk, w1, b1.reshape(1, D), w2.reshape(1, D), b2.reshape(1, 1))


def kernel(x, edge_index, batch,
           c1_Wa, c1_ba, c1_Wb, c1_bb, bn1_g, bn1_b,
           c2_Wa, c2_ba, c2_Wb, c2_bb, bn2_g, bn2_b,
           c3_Wa, c3_ba, c3_Wb, c3_bb, bn3_g, bn3_b,
           fc1_W, fc1_b, fc2_W, fc2_b):
  pad_idx = jnp.asarray(_PAD_IDX)
  src = jnp.concatenate(
      [edge_index[0].astype(jnp.int32), pad_idx]).reshape(ER, 128)
  dst = jnp.concatenate(
      [edge_index[1].astype(jnp.int32), pad_idx]).reshape(ER, 128)
  bpad = jnp.concatenate(
      [batch.astype(jnp.int32), jnp.full((NP - N,), G - 1, jnp.int32)]
  ).reshape(NPR, 128)
  xpk = jnp.pad(x, ((0, NP - N), (0, H - 3))).reshape(PR, 128)

  agg = _agg_call()
  y = _embed(xpk, c1_Wa)
  a = agg(y.reshape(NC * NP, H), src, dst).reshape(2, PR, 128)
  y = _layer(y, a, c1_ba, c1_Wb, c1_bb, bn1_g, bn1_b, c2_Wa)
  a = agg(y.reshape(NC * NP, H), src, dst).reshape(2, PR, 128)
  y = _layer(y, a, c2_ba, c2_Wb, c2_bb, bn2_g, bn2_b, c3_Wa)
  a = agg(y.reshape(NC * NP, H), src, dst).reshape(2, PR, 128)
  h3 = _final(y, a, c3_ba, c3_Wb, c3_bb, bn3_g, bn3_b)
  pooled = _pool_call()(h3.reshape(NC * NP, H), bpad).reshape(2, GP // 8, 128)
  out = _head(pooled, fc1_W, fc1_b, fc2_W, fc2_b)
  return out.reshape(GP, 1)[:G]


# speedup vs baseline: 13.9704x; 1.1219x over previous
import functools

import jax
import jax.numpy as jnp
import numpy as np
from jax import lax
from jax.experimental import pallas as pl
from jax.experimental.pallas import tpu as pltpu
from jax.experimental.pallas import tpu_sc as plsc

N = 100000
E = 1600000
G = 1000
GP = 1024
D = 32
H = 16
NP = 102400
EP = 1638400
ER = EP // 128
NPR = NP // 128
NC = 2
NS = 16
R = 1024
BN_SCALE = float(1.0 / (1.0 + 1e-5) ** 0.5)

ETR = ER // NS
SUP = 4
NSUP = ETR // SUP

_PAD_IDX = np.asarray(N + np.arange(EP - E) % 2048, dtype=np.int32)

_sc_mesh = functools.partial(
    plsc.VectorSubcoreMesh,
    core_axis_name="c", subcore_axis_name="s", num_cores=NC, num_subcores=NS,
)
_SC_PARAMS = pltpu.CompilerParams(use_tc_tiling_on_sc=False)


def _agg_body(y_hbm, src_hbm, dst_hbm, out_hbm, acc, sidx, didx, rows, zbuf,
              gsem0, gsem1, ssem0, ssem1):
  c = lax.axis_index("c")
  s = lax.axis_index("s")
  gsems = (gsem0, gsem1)
  ssems = (ssem0, ssem1)

  def zfill(i, carry):
    zbuf[i] = jnp.zeros((H,), jnp.float32)
    return carry
  lax.fori_loop(0, 256, zfill, 0)
  zrow0 = s * (NP // NS)
  for k in range(25):
    pltpu.sync_copy(zbuf, acc.at[pl.ds(zrow0 + k * 256, 256)])
  plsc.subcore_barrier()

  coff = c * NP
  r0 = s * ETR

  def load_and_gather(u, k):
    jrow = r0 + u * SUP
    pltpu.sync_copy(src_hbm.at[pl.ds(jrow, SUP)], sidx.at[k])
    pltpu.sync_copy(dst_hbm.at[pl.ds(jrow, SUP)], didx.at[k])
    for r in range(SUP):
      for q in range(8):
        sidx[k, r, pl.ds(q * 16, 16)] = sidx[k, r, pl.ds(q * 16, 16)] + coff
    for r in range(SUP):
      pltpu.async_copy(y_hbm.at[sidx.at[k].at[r]], rows.at[k].at[r], gsems[k])

  def wait_gathers_fire_scatters(k):
    for r in range(SUP):
      pltpu.make_async_copy(
          y_hbm.at[sidx.at[k].at[r]], rows.at[k].at[r], gsems[k]).wait()
    for r in range(SUP):
      pltpu.async_copy(rows.at[k].at[r], acc.at[didx.at[k].at[r]], ssems[k],
                       add=True)

  def wait_scatters(k):
    for r in range(SUP):
      pltpu.make_async_copy(
          rows.at[k].at[r], acc.at[didx.at[k].at[r]], ssems[k]).wait()

  load_and_gather(0, 0)
  load_and_gather(1, 1)

  def pipe(h, carry):
    wait_gathers_fire_scatters(0)
    wait_gathers_fire_scatters(1)
    wait_scatters(0)
    load_and_gather(2 * h + 2, 0)
    wait_scatters(1)
    load_and_gather(2 * h + 3, 1)
    return carry

  lax.fori_loop(0, NSUP // 2 - 1, pipe, 0)
  wait_gathers_fire_scatters(0)
  wait_gathers_fire_scatters(1)
  wait_scatters(0)
  wait_scatters(1)
  plsc.subcore_barrier()
  wrow0 = s * (NP // NS)
  pltpu.sync_copy(acc.at[pl.ds(wrow0, NP // NS)],
                  out_hbm.at[pl.ds(coff + wrow0, NP // NS)])


@functools.cache
def _agg_call():
  return pl.kernel(
      _agg_body,
      out_type=jax.ShapeDtypeStruct((NC * NP, H), jnp.float32),
      mesh=_sc_mesh(),
      scratch_types=[
          pltpu.VMEM_SHARED((NP, H), jnp.float32),
          pltpu.VMEM((2, SUP, 128), jnp.int32),
          pltpu.VMEM((2, SUP, 128), jnp.int32),
          pltpu.VMEM((2, SUP, 128, H), jnp.float32),
          pltpu.VMEM((256, H), jnp.float32),
          pltpu.SemaphoreType.DMA,
          pltpu.SemaphoreType.DMA,
          pltpu.SemaphoreType.DMA,
          pltpu.SemaphoreType.DMA,
      ],
      compiler_params=_SC_PARAMS,
  )


def _pool_body(h_hbm, b_hbm, out_hbm, acc, bidx, vrows, zbuf, gsem):
  c = lax.axis_index("c")
  s = lax.axis_index("s")

  def zfill(i, carry):
    zbuf[i] = jnp.zeros((H,), jnp.float32)
    return carry
  lax.fori_loop(0, 64, zfill, 0)
  pltpu.sync_copy(zbuf, acc.at[pl.ds(s * 64, 64)])
  plsc.subcore_barrier()

  def stage(u, carry):
    g = s + u * NS
    pltpu.sync_copy(b_hbm.at[pl.ds(g * 8, 8)], bidx)
    pltpu.sync_copy(h_hbm.at[pl.ds(c * NP + g * 1024, 1024)], vrows)
    for r in range(8):
      pltpu.sync_copy(vrows.at[pl.ds(r * 128, 128)], acc.at[bidx.at[r]],
                      add=True)
    return carry
  lax.fori_loop(0, 6 + (s < 4).astype(jnp.int32), stage, 0)

  plsc.subcore_barrier()
  pltpu.sync_copy(acc.at[pl.ds(s * 64, 64)],
                  out_hbm.at[pl.ds(c * GP + s * 64, 64)])


@functools.cache
def _pool_call():
  return pl.kernel(
      _pool_body,
      out_type=jax.ShapeDtypeStruct((NC * GP, H), jnp.float32),
      mesh=_sc_mesh(),
      scratch_types=[
          pltpu.VMEM_SHARED((GP, H), jnp.float32),
          pltpu.VMEM((8, 128), jnp.int32),
          pltpu.VMEM((1024, H), jnp.float32),
          pltpu.VMEM((64, H), jnp.float32),
          pltpu.SemaphoreType.DMA,
      ],
      compiler_params=_SC_PARAMS,
  )


PR = NP // 8
RB = 1280


def _embed_body(x_ref, w_ref, o_ref):
  w = w_ref[...]
  z0, z1 = [], []
  for j in range(8):
    xj = x_ref[:, 16 * j:16 * j + 3]
    y = (xj[:, 0:1] * w[0:1, :] + xj[:, 1:2] * w[1:2, :]
         + xj[:, 2:3] * w[2:3, :])
    z0.append(y[:, :H])
    z1.append(y[:, H:])
  o_ref[0] = jnp.concatenate(z0, axis=1)
  o_ref[1] = jnp.concatenate(z1, axis=1)


def _embed(xpk, wa):
  return pl.pallas_call(
      _embed_body,
      grid=(PR // RB,),
      in_specs=[
          pl.BlockSpec((RB, 128), lambda i: (i, 0)),
          pl.BlockSpec((3, D), lambda i: (0, 0)),
      ],
      out_specs=pl.BlockSpec((2, RB, 128), lambda i: (0, i, 0)),
      out_shape=jax.ShapeDtypeStruct((2, PR, 128), jnp.float32),
  )(xpk, wa)


def _mlp_slice(y_ref, a_ref, j, ba, wb, bb, g2, b2):
  lo, hi = 16 * j, 16 * j + 16
  h = jnp.concatenate(
      [y_ref[0][:, lo:hi] + a_ref[0][:, lo:hi],
       y_ref[1][:, lo:hi] + a_ref[1][:, lo:hi]],
      axis=1)
  h = jnp.maximum(h + ba, 0.0)
  u = jnp.maximum(
      jnp.dot(h, wb, preferred_element_type=jnp.float32) + bb, 0.0)
  return u * g2 + b2


def _layer_body(y_ref, a_ref, ba_ref, wb_ref, bb_ref, g_ref, b_ref, wn_ref,
                o_ref):
  ba = ba_ref[...]
  wb = wb_ref[...]
  bb = bb_ref[...]
  g2 = g_ref[...] * BN_SCALE
  b2 = b_ref[...]
  wn = wn_ref[...]
  z0, z1 = [], []
  for j in range(8):
    v = _mlp_slice(y_ref, a_ref, j, ba, wb, bb, g2, b2)
    z = jnp.dot(v, wn, preferred_element_type=jnp.float32)
    z0.append(z[:, :H])
    z1.append(z[:, H:])
  o_ref[0] = jnp.concatenate(z0, axis=1)
  o_ref[1] = jnp.concatenate(z1, axis=1)


def _final_body(y_ref, a_ref, ba_ref, wb_ref, bb_ref, g_ref, b_ref, o_ref):
  i = pl.program_id(0)
  ba = ba_ref[...]
  wb = wb_ref[...]
  bb = bb_ref[...]
  g2 = g_ref[...] * BN_SCALE
  b2 = b_ref[...]
  row = i * RB + lax.broadcasted_iota(jnp.int32, (RB, 1), 0)
  z0, z1 = [], []
  for j in range(8):
    v = _mlp_slice(y_ref, a_ref, j, ba, wb, bb, g2, b2)
    v = jnp.where(row * 8 + j < N, v, 0.0)
    z0.append(v[:, :H])
    z1.append(v[:, H:])
  o_ref[0] = jnp.concatenate(z0, axis=1)
  o_ref[1] = jnp.concatenate(z1, axis=1)


def _split_specs():
  return [
      pl.BlockSpec((2, RB, 128), lambda i: (0, i, 0)),
      pl.BlockSpec((2, RB, 128), lambda i: (0, i, 0)),
      pl.BlockSpec((1, D), lambda i: (0, 0)),
      pl.BlockSpec((D, D), lambda i: (0, 0)),
      pl.BlockSpec((1, D), lambda i: (0, 0)),
      pl.BlockSpec((1, D), lambda i: (0, 0)),
      pl.BlockSpec((1, D), lambda i: (0, 0)),
  ]


def _layer(y, a, ba, wb, bb, g, b, wn):
  return pl.pallas_call(
      _layer_body,
      grid=(PR // RB,),
      in_specs=_split_specs() + [pl.BlockSpec((D, D), lambda i: (0, 0))],
      out_specs=pl.BlockSpec((2, RB, 128), lambda i: (0, i, 0)),
      out_shape=jax.ShapeDtypeStruct((2, PR, 128), jnp.float32),
  )(y, a, ba.reshape(1, D), wb, bb.reshape(1, D), g.reshape(1, D),
    b.reshape(1, D), wn)


def _final(y, a, ba, wb, bb, g, b):
  return pl.pallas_call(
      _final_body,
      grid=(PR // RB,),
      in_specs=_split_specs(),
      out_specs=pl.BlockSpec((2, RB, 128), lambda i: (0, i, 0)),
      out_shape=jax.ShapeDtypeStruct((2, PR, 128), jnp.float32),
  )(y, a, ba.reshape(1, D), wb, bb.reshape(1, D), g.reshape(1, D),
    b.reshape(1, D))


def _head_body(p_ref, w1_ref, b1_ref, w2_ref, b2_ref, o_ref):
  w1 = w1_ref[...]
  b1 = b1_ref[...]
  w2 = w2_ref[...]
  cols = []
  for j in range(8):
    lo, hi = 16 * j, 16 * j + 16
    p = jnp.concatenate([p_ref[0][:, lo:hi], p_ref[1][:, lo:hi]], axis=1)
    h = jnp.maximum(
        jnp.dot(p, w1, preferred_element_type=jnp.float32) + b1, 0.0)
    cols.append(jnp.sum(h * w2, axis=1, keepdims=True))
  o = jnp.concatenate(cols, axis=1) + b2_ref[...]
  o_ref[...] = jnp.tanh(o)


def _head(pooled_pk, w1, b1, w2, b2):
  return pl.pallas_call(
      _head_body,
      out_shape=jax.ShapeDtypeStruct((GP // 8, 8), jnp.float32),
  )(pooled_pk, w1, b1.reshape(1, D), w2.reshape(1, D), b2.reshape(1, 1))


def kernel(x, edge_index, batch,
           c1_Wa, c1_ba, c1_Wb, c1_bb, bn1_g, bn1_b,
           c2_Wa, c2_ba, c2_Wb, c2_bb, bn2_g, bn2_b,
           c3_Wa, c3_ba, c3_Wb, c3_bb, bn3_g, bn3_b,
           fc1_W, fc1_b, fc2_W, fc2_b):
  pad_idx = jnp.asarray(_PAD_IDX)
  src = jnp.concatenate(
      [edge_index[0].astype(jnp.int32), pad_idx]).reshape(ER, 128)
  dst = jnp.concatenate(
      [edge_index[1].astype(jnp.int32), pad_idx]).reshape(ER, 128)
  bpad = jnp.concatenate(
      [batch.astype(jnp.int32), jnp.full((NP - N,), G - 1, jnp.int32)]
  ).reshape(NPR, 128)
  xpk = jnp.pad(x, ((0, NP - N), (0, H - 3))).reshape(PR, 128)

  agg = _agg_call()
  y = _embed(xpk, c1_Wa)
  a = agg(y.reshape(NC * NP, H), src, dst).reshape(2, PR, 128)
  y = _layer(y, a, c1_ba, c1_Wb, c1_bb, bn1_g, bn1_b, c2_Wa)
  a = agg(y.reshape(NC * NP, H), src, dst).reshape(2, PR, 128)
  y = _layer(y, a, c2_ba, c2_Wb, c2_bb, bn2_g, bn2_b, c3_Wa)
  a = agg(y.reshape(NC * NP, H), src, dst).reshape(2, PR, 128)
  h3 = _final(y, a, c3_ba, c3_Wb, c3_bb, bn3_g, bn3_b)
  pooled = _pool_call()(h3.reshape(NC * NP, H), bpad).reshape(2, GP // 8, 128)
  out = _head(pooled, fc1_W, fc1_b, fc2_W, fc2_b)
  return out.reshape(GP, 1)[:G]


# speedup vs baseline: 14.0325x; 1.0044x over previous
import functools

import jax
import jax.numpy as jnp
import numpy as np
from jax import lax
from jax.experimental import pallas as pl
from jax.experimental.pallas import tpu as pltpu
from jax.experimental.pallas import tpu_sc as plsc

N = 100000
E = 1600000
G = 1000
GP = 1024
D = 32
H = 16
NP = 102400
EP = 1638400
ER = EP // 128
NPR = NP // 128
NC = 2
NS = 16
R = 1024
BN_SCALE = float(1.0 / (1.0 + 1e-5) ** 0.5)

ETR = ER // NS
CHUNK = 512
NSUP = ETR * 128 // CHUNK

_PAD_IDX = np.asarray(N + np.arange(EP - E) % 2048, dtype=np.int32)

_sc_mesh = functools.partial(
    plsc.VectorSubcoreMesh,
    core_axis_name="c", subcore_axis_name="s", num_cores=NC, num_subcores=NS,
)
_SC_PARAMS = pltpu.CompilerParams(use_tc_tiling_on_sc=False)


def _agg_body(y_hbm, src_hbm, dst_hbm, out_hbm, acc, sidx, didx, rows, zbuf,
              gsem0, gsem1, ssem0, ssem1):
  c = lax.axis_index("c")
  s = lax.axis_index("s")
  gsems = (gsem0, gsem1)
  ssems = (ssem0, ssem1)

  def zfill(i, carry):
    zbuf[i] = jnp.zeros((H,), jnp.float32)
    return carry
  lax.fori_loop(0, 256, zfill, 0)
  zrow0 = s * (NP // NS)
  for k in range(25):
    pltpu.sync_copy(zbuf, acc.at[pl.ds(zrow0 + k * 256, 256)])
  plsc.subcore_barrier()

  coff = c * NP
  e0 = s * ETR * 128

  def load_and_gather(u, k):
    base = e0 + u * CHUNK
    pltpu.sync_copy(src_hbm.at[pl.ds(base, CHUNK)], sidx.at[k])
    pltpu.sync_copy(dst_hbm.at[pl.ds(base, CHUNK)], didx.at[k])
    for q in range(CHUNK // 16):
      sidx[k, pl.ds(q * 16, 16)] = sidx[k, pl.ds(q * 16, 16)] + coff
    pltpu.async_copy(y_hbm.at[sidx.at[k]], rows.at[k], gsems[k])

  def wait_gathers_fire_scatters(k):
    pltpu.make_async_copy(y_hbm.at[sidx.at[k]], rows.at[k], gsems[k]).wait()
    pltpu.async_copy(rows.at[k], acc.at[didx.at[k]], ssems[k], add=True)

  def wait_scatters(k):
    pltpu.make_async_copy(rows.at[k], acc.at[didx.at[k]], ssems[k]).wait()

  load_and_gather(0, 0)
  load_and_gather(1, 1)

  def pipe(h, carry):
    wait_gathers_fire_scatters(0)
    wait_gathers_fire_scatters(1)
    wait_scatters(0)
    load_and_gather(2 * h + 2, 0)
    wait_scatters(1)
    load_and_gather(2 * h + 3, 1)
    return carry

  lax.fori_loop(0, NSUP // 2 - 1, pipe, 0)
  wait_gathers_fire_scatters(0)
  wait_gathers_fire_scatters(1)
  wait_scatters(0)
  wait_scatters(1)
  plsc.subcore_barrier()
  wrow0 = s * (NP // NS)
  pltpu.sync_copy(acc.at[pl.ds(wrow0, NP // NS)],
                  out_hbm.at[pl.ds(coff + wrow0, NP // NS)])


@functools.cache
def _agg_call():
  return pl.kernel(
      _agg_body,
      out_type=jax.ShapeDtypeStruct((NC * NP, H), jnp.float32),
      mesh=_sc_mesh(),
      scratch_types=[
          pltpu.VMEM_SHARED((NP, H), jnp.float32),
          pltpu.VMEM((2, CHUNK), jnp.int32),
          pltpu.VMEM((2, CHUNK), jnp.int32),
          pltpu.VMEM((2, CHUNK, H), jnp.float32),
          pltpu.VMEM((256, H), jnp.float32),
          pltpu.SemaphoreType.DMA,
          pltpu.SemaphoreType.DMA,
          pltpu.SemaphoreType.DMA,
          pltpu.SemaphoreType.DMA,
      ],
      compiler_params=_SC_PARAMS,
  )


def _pool_body(h_hbm, b_hbm, out_hbm, acc, bidx, vrows, zbuf, gsem):
  c = lax.axis_index("c")
  s = lax.axis_index("s")

  def zfill(i, carry):
    zbuf[i] = jnp.zeros((H,), jnp.float32)
    return carry
  lax.fori_loop(0, 64, zfill, 0)
  pltpu.sync_copy(zbuf, acc.at[pl.ds(s * 64, 64)])
  plsc.subcore_barrier()

  def stage(u, carry):
    g = s + u * NS
    pltpu.sync_copy(b_hbm.at[pl.ds(g * 8, 8)], bidx)
    pltpu.sync_copy(h_hbm.at[pl.ds(c * NP + g * 1024, 1024)], vrows)
    for r in range(8):
      pltpu.sync_copy(vrows.at[pl.ds(r * 128, 128)], acc.at[bidx.at[r]],
                      add=True)
    return carry
  lax.fori_loop(0, 6 + (s < 4).astype(jnp.int32), stage, 0)

  plsc.subcore_barrier()
  pltpu.sync_copy(acc.at[pl.ds(s * 64, 64)],
                  out_hbm.at[pl.ds(c * GP + s * 64, 64)])


@functools.cache
def _pool_call():
  return pl.kernel(
      _pool_body,
      out_type=jax.ShapeDtypeStruct((NC * GP, H), jnp.float32),
      mesh=_sc_mesh(),
      scratch_types=[
          pltpu.VMEM_SHARED((GP, H), jnp.float32),
          pltpu.VMEM((8, 128), jnp.int32),
          pltpu.VMEM((1024, H), jnp.float32),
          pltpu.VMEM((64, H), jnp.float32),
          pltpu.SemaphoreType.DMA,
      ],
      compiler_params=_SC_PARAMS,
  )


PR = NP // 8
RB = 1280


def _embed_body(x_ref, w_ref, o_ref):
  w = w_ref[...]
  z0, z1 = [], []
  for j in range(8):
    xj = x_ref[:, 16 * j:16 * j + 3]
    y = (xj[:, 0:1] * w[0:1, :] + xj[:, 1:2] * w[1:2, :]
         + xj[:, 2:3] * w[2:3, :])
    z0.append(y[:, :H])
    z1.append(y[:, H:])
  o_ref[0] = jnp.concatenate(z0, axis=1)
  o_ref[1] = jnp.concatenate(z1, axis=1)


def _embed(xpk, wa):
  return pl.pallas_call(
      _embed_body,
      grid=(PR // RB,),
      in_specs=[
          pl.BlockSpec((RB, 128), lambda i: (i, 0)),
          pl.BlockSpec((3, D), lambda i: (0, 0)),
      ],
      out_specs=pl.BlockSpec((2, RB, 128), lambda i: (0, i, 0)),
      out_shape=jax.ShapeDtypeStruct((2, PR, 128), jnp.float32),
  )(xpk, wa)


def _mlp_slice(y_ref, a_ref, j, ba, wb, bb, g2, b2):
  lo, hi = 16 * j, 16 * j + 16
  h = jnp.concatenate(
      [y_ref[0][:, lo:hi] + a_ref[0][:, lo:hi],
       y_ref[1][:, lo:hi] + a_ref[1][:, lo:hi]],
      axis=1)
  h = jnp.maximum(h + ba, 0.0)
  u = jnp.maximum(
      jnp.dot(h, wb, preferred_element_type=jnp.float32) + bb, 0.0)
  return u * g2 + b2


def _layer_body(y_ref, a_ref, ba_ref, wb_ref, bb_ref, g_ref, b_ref, wn_ref,
                o_ref):
  ba = ba_ref[...]
  wb = wb_ref[...]
  bb = bb_ref[...]
  g2 = g_ref[...] * BN_SCALE
  b2 = b_ref[...]
  wn = wn_ref[...]
  z0, z1 = [], []
  for j in range(8):
    v = _mlp_slice(y_ref, a_ref, j, ba, wb, bb, g2, b2)
    z = jnp.dot(v, wn, preferred_element_type=jnp.float32)
    z0.append(z[:, :H])
    z1.append(z[:, H:])
  o_ref[0] = jnp.concatenate(z0, axis=1)
  o_ref[1] = jnp.concatenate(z1, axis=1)


def _final_body(y_ref, a_ref, ba_ref, wb_ref, bb_ref, g_ref, b_ref, o_ref):
  i = pl.program_id(0)
  ba = ba_ref[...]
  wb = wb_ref[...]
  bb = bb_ref[...]
  g2 = g_ref[...] * BN_SCALE
  b2 = b_ref[...]
  row = i * RB + lax.broadcasted_iota(jnp.int32, (RB, 1), 0)
  z0, z1 = [], []
  for j in range(8):
    v = _mlp_slice(y_ref, a_ref, j, ba, wb, bb, g2, b2)
    v = jnp.where(row * 8 + j < N, v, 0.0)
    z0.append(v[:, :H])
    z1.append(v[:, H:])
  o_ref[0] = jnp.concatenate(z0, axis=1)
  o_ref[1] = jnp.concatenate(z1, axis=1)


def _split_specs():
  return [
      pl.BlockSpec((2, RB, 128), lambda i: (0, i, 0)),
      pl.BlockSpec((2, RB, 128), lambda i: (0, i, 0)),
      pl.BlockSpec((1, D), lambda i: (0, 0)),
      pl.BlockSpec((D, D), lambda i: (0, 0)),
      pl.BlockSpec((1, D), lambda i: (0, 0)),
      pl.BlockSpec((1, D), lambda i: (0, 0)),
      pl.BlockSpec((1, D), lambda i: (0, 0)),
  ]


def _layer(y, a, ba, wb, bb, g, b, wn):
  return pl.pallas_call(
      _layer_body,
      grid=(PR // RB,),
      in_specs=_split_specs() + [pl.BlockSpec((D, D), lambda i: (0, 0))],
      out_specs=pl.BlockSpec((2, RB, 128), lambda i: (0, i, 0)),
      out_shape=jax.ShapeDtypeStruct((2, PR, 128), jnp.float32),
  )(y, a, ba.reshape(1, D), wb, bb.reshape(1, D), g.reshape(1, D),
    b.reshape(1, D), wn)


def _final(y, a, ba, wb, bb, g, b):
  return pl.pallas_call(
      _final_body,
      grid=(PR // RB,),
      in_specs=_split_specs(),
      out_specs=pl.BlockSpec((2, RB, 128), lambda i: (0, i, 0)),
      out_shape=jax.ShapeDtypeStruct((2, PR, 128), jnp.float32),
  )(y, a, ba.reshape(1, D), wb, bb.reshape(1, D), g.reshape(1, D),
    b.reshape(1, D))


def _head_body(p_ref, w1_ref, b1_ref, w2_ref, b2_ref, o_ref):
  w1 = w1_ref[...]
  b1 = b1_ref[...]
  w2 = w2_ref[...]
  cols = []
  for j in range(8):
    lo, hi = 16 * j, 16 * j + 16
    p = jnp.concatenate([p_ref[0][:, lo:hi], p_ref[1][:, lo:hi]], axis=1)
    h = jnp.maximum(
        jnp.dot(p, w1, preferred_element_type=jnp.float32) + b1, 0.0)
    cols.append(jnp.sum(h * w2, axis=1, keepdims=True))
  o = jnp.concatenate(cols, axis=1) + b2_ref[...]
  o_ref[...] = jnp.tanh(o)


def _head(pooled_pk, w1, b1, w2, b2):
  return pl.pallas_call(
      _head_body,
      out_shape=jax.ShapeDtypeStruct((GP // 8, 8), jnp.float32),
  )(pooled_pk, w1, b1.reshape(1, D), w2.reshape(1, D), b2.reshape(1, 1))


def kernel(x, edge_index, batch,
           c1_Wa, c1_ba, c1_Wb, c1_bb, bn1_g, bn1_b,
           c2_Wa, c2_ba, c2_Wb, c2_bb, bn2_g, bn2_b,
           c3_Wa, c3_ba, c3_Wb, c3_bb, bn3_g, bn3_b,
           fc1_W, fc1_b, fc2_W, fc2_b):
  pad_idx = jnp.asarray(_PAD_IDX)
  src = jnp.concatenate([edge_index[0].astype(jnp.int32), pad_idx])
  dst = jnp.concatenate([edge_index[1].astype(jnp.int32), pad_idx])
  bpad = jnp.concatenate(
      [batch.astype(jnp.int32), jnp.full((NP - N,), G - 1, jnp.int32)]
  ).reshape(NPR, 128)
  xpk = jnp.pad(x, ((0, NP - N), (0, H - 3))).reshape(PR, 128)

  agg = _agg_call()
  y = _embed(xpk, c1_Wa)
  a = agg(y.reshape(NC * NP, H), src, dst).reshape(2, PR, 128)
  y = _layer(y, a, c1_ba, c1_Wb, c1_bb, bn1_g, bn1_b, c2_Wa)
  a = agg(y.reshape(NC * NP, H), src, dst).reshape(2, PR, 128)
  y = _layer(y, a, c2_ba, c2_Wb, c2_bb, bn2_g, bn2_b, c3_Wa)
  a = agg(y.reshape(NC * NP, H), src, dst).reshape(2, PR, 128)
  h3 = _final(y, a, c3_ba, c3_Wb, c3_bb, bn3_g, bn3_b)
  pooled = _pool_call()(h3.reshape(NC * NP, H), bpad).reshape(2, GP // 8, 128)
  out = _head(pooled, fc1_W, fc1_b, fc2_W, fc2_b)
  return out.reshape(GP, 1)[:G]


# speedup vs baseline: 16.1765x; 1.1528x over previous
import functools

import jax
import jax.numpy as jnp
import numpy as np
from jax import lax
from jax.experimental import pallas as pl
from jax.experimental.pallas import tpu as pltpu
from jax.experimental.pallas import tpu_sc as plsc

N = 100000
E = 1600000
G = 1000
GP = 1024
D = 32
H = 16
NP = 102400
NPR = NP // 128
NC = 2
NS = 16
R = 1024
BN_SCALE = float(1.0 / (1.0 + 1e-5) ** 0.5)

CHUNK = 512
ETILE = 102400
NSUP = ETILE // CHUNK
NSUP_LAST = (E - 15 * ETILE) // CHUNK

_sc_mesh = functools.partial(
    plsc.VectorSubcoreMesh,
    core_axis_name="c", subcore_axis_name="s", num_cores=NC, num_subcores=NS,
)
_SC_PARAMS = pltpu.CompilerParams(use_tc_tiling_on_sc=False)


def _agg_body(y_hbm, edges_hbm, out_hbm, acc, sidx, didx, rows, zbuf,
              gsem0, gsem1, ssem0, ssem1):
  c = lax.axis_index("c")
  s = lax.axis_index("s")
  gsems = (gsem0, gsem1)
  ssems = (ssem0, ssem1)

  def zfill(i, carry):
    zbuf[i] = jnp.zeros((H,), jnp.float32)
    return carry
  lax.fori_loop(0, 256, zfill, 0)
  zrow0 = s * (NP // NS)
  for k in range(25):
    pltpu.sync_copy(zbuf, acc.at[pl.ds(zrow0 + k * 256, 256)])
  plsc.subcore_barrier()

  coff = c * NP
  e0 = s * ETILE
  npairs_m1 = jnp.where(s == NS - 1, (NSUP_LAST - 1) // 2 - 1, NSUP // 2 - 1)

  def load_and_gather(u, k):
    base = e0 + u * CHUNK
    pltpu.sync_copy(edges_hbm.at[pl.ds(base, CHUNK)], sidx.at[k])
    pltpu.sync_copy(edges_hbm.at[pl.ds(E + base, CHUNK)], didx.at[k])
    for q in range(CHUNK // 16):
      sidx[k, pl.ds(q * 16, 16)] = sidx[k, pl.ds(q * 16, 16)] + coff
    pltpu.async_copy(y_hbm.at[sidx.at[k]], rows.at[k], gsems[k])

  def wait_gathers_fire_scatters(k):
    pltpu.make_async_copy(y_hbm.at[sidx.at[k]], rows.at[k], gsems[k]).wait()
    pltpu.async_copy(rows.at[k], acc.at[didx.at[k]], ssems[k], add=True)

  def wait_scatters(k):
    pltpu.make_async_copy(rows.at[k], acc.at[didx.at[k]], ssems[k]).wait()

  load_and_gather(0, 0)
  load_and_gather(1, 1)

  def pipe(h, carry):
    wait_gathers_fire_scatters(0)
    wait_gathers_fire_scatters(1)
    wait_scatters(0)
    load_and_gather(2 * h + 2, 0)
    wait_scatters(1)
    load_and_gather(2 * h + 3, 1)
    return carry

  lax.fori_loop(0, npairs_m1, pipe, 0)
  wait_gathers_fire_scatters(0)
  wait_gathers_fire_scatters(1)
  wait_scatters(0)
  wait_scatters(1)

  @pl.when(s == NS - 1)
  def _odd_tail():
    load_and_gather(NSUP_LAST - 1, 0)
    wait_gathers_fire_scatters(0)
    wait_scatters(0)

  plsc.subcore_barrier()
  wrow0 = s * (NP // NS)
  pltpu.sync_copy(acc.at[pl.ds(wrow0, NP // NS)],
                  out_hbm.at[pl.ds(coff + wrow0, NP // NS)])


@functools.cache
def _agg_call():
  return pl.kernel(
      _agg_body,
      out_type=jax.ShapeDtypeStruct((NC * NP, H), jnp.float32),
      mesh=_sc_mesh(),
      scratch_types=[
          pltpu.VMEM_SHARED((NP, H), jnp.float32),
          pltpu.VMEM((2, CHUNK), jnp.int32),
          pltpu.VMEM((2, CHUNK), jnp.int32),
          pltpu.VMEM((2, CHUNK, H), jnp.float32),
          pltpu.VMEM((256, H), jnp.float32),
          pltpu.SemaphoreType.DMA,
          pltpu.SemaphoreType.DMA,
          pltpu.SemaphoreType.DMA,
          pltpu.SemaphoreType.DMA,
      ],
      compiler_params=_SC_PARAMS,
  )


def _pool_body(h_hbm, b_hbm, out_hbm, acc, bidx, vrows, zbuf, gsem):
  c = lax.axis_index("c")
  s = lax.axis_index("s")

  def zfill(i, carry):
    zbuf[i] = jnp.zeros((H,), jnp.float32)
    return carry
  lax.fori_loop(0, 64, zfill, 0)
  pltpu.sync_copy(zbuf, acc.at[pl.ds(s * 64, 64)])
  plsc.subcore_barrier()

  def stage(u, carry):
    g = s + u * NS
    pltpu.sync_copy(b_hbm.at[pl.ds(g * 8, 8)], bidx)
    pltpu.sync_copy(h_hbm.at[pl.ds(c * NP + g * 1024, 1024)], vrows)
    for r in range(8):
      pltpu.sync_copy(vrows.at[pl.ds(r * 128, 128)], acc.at[bidx.at[r]],
                      add=True)
    return carry
  lax.fori_loop(0, 6 + (s < 4).astype(jnp.int32), stage, 0)

  plsc.subcore_barrier()
  pltpu.sync_copy(acc.at[pl.ds(s * 64, 64)],
                  out_hbm.at[pl.ds(c * GP + s * 64, 64)])


@functools.cache
def _pool_call():
  return pl.kernel(
      _pool_body,
      out_type=jax.ShapeDtypeStruct((NC * GP, H), jnp.float32),
      mesh=_sc_mesh(),
      scratch_types=[
          pltpu.VMEM_SHARED((GP, H), jnp.float32),
          pltpu.VMEM((8, 128), jnp.int32),
          pltpu.VMEM((1024, H), jnp.float32),
          pltpu.VMEM((64, H), jnp.float32),
          pltpu.SemaphoreType.DMA,
      ],
      compiler_params=_SC_PARAMS,
  )


PR = NP // 8
RB = 1280


def _blk(m16):
  return jnp.kron(jnp.eye(8, dtype=jnp.float32), m16)


def _quads(w):
  return jnp.stack([_blk(w[:H, :H]), _blk(w[H:, :H]),
                    _blk(w[:H, H:]), _blk(w[H:, H:])])


def _tile8(v16):
  return jnp.tile(v16, 8).reshape(1, 128)


def _embed_body(x_ref, a_ref, o_ref):
  xb = x_ref[...]
  o_ref[0] = jnp.dot(xb, a_ref[0], preferred_element_type=jnp.float32)
  o_ref[1] = jnp.dot(xb, a_ref[1], preferred_element_type=jnp.float32)


def _embed(xpk, a2):
  return pl.pallas_call(
      _embed_body,
      grid=(PR // RB,),
      in_specs=[
          pl.BlockSpec((RB, 128), lambda i: (i, 0)),
          pl.BlockSpec((2, 128, 128), lambda i: (0, 0, 0)),
      ],
      out_specs=pl.BlockSpec((2, RB, 128), lambda i: (0, i, 0)),
      out_shape=jax.ShapeDtypeStruct((2, PR, 128), jnp.float32),
  )(xpk, a2)


def _mlp_packed(y_ref, a_ref, wu, bv):
  h0 = jnp.maximum(y_ref[0] + a_ref[0] + bv[0:1], 0.0)
  h1 = jnp.maximum(y_ref[1] + a_ref[1] + bv[1:2], 0.0)
  u0 = jnp.dot(h0, wu[0], preferred_element_type=jnp.float32)
  u0 += jnp.dot(h1, wu[1], preferred_element_type=jnp.float32)
  u1 = jnp.dot(h0, wu[2], preferred_element_type=jnp.float32)
  u1 += jnp.dot(h1, wu[3], preferred_element_type=jnp.float32)
  u0 = jnp.maximum(u0 + bv[2:3], 0.0)
  u1 = jnp.maximum(u1 + bv[3:4], 0.0)
  v0 = u0 * bv[4:5] + bv[6:7]
  v1 = u1 * bv[5:6] + bv[7:8]
  return v0, v1


def _layer_body(y_ref, a_ref, wu_ref, wz_ref, bv_ref, o_ref):
  wu = wu_ref[...]
  wz = wz_ref[...]
  v0, v1 = _mlp_packed(y_ref, a_ref, wu, bv_ref[...])
  z0 = jnp.dot(v0, wz[0], preferred_element_type=jnp.float32)
  z0 += jnp.dot(v1, wz[1], preferred_element_type=jnp.float32)
  z1 = jnp.dot(v0, wz[2], preferred_element_type=jnp.float32)
  z1 += jnp.dot(v1, wz[3], preferred_element_type=jnp.float32)
  o_ref[0] = z0
  o_ref[1] = z1


def _final_body(y_ref, a_ref, wu_ref, bv_ref, o_ref):
  i = pl.program_id(0)
  v0, v1 = _mlp_packed(y_ref, a_ref, wu_ref[...], bv_ref[...])
  row = i * RB + lax.broadcasted_iota(jnp.int32, (RB, 128), 0)
  node = row * 8 + lax.broadcasted_iota(jnp.int32, (RB, 128), 1) // H
  keep = node < N
  o_ref[0] = jnp.where(keep, v0, 0.0)
  o_ref[1] = jnp.where(keep, v1, 0.0)


def _mk_bv(ba, bb, g, b):
  return jnp.stack([
      jnp.tile(ba[:H], 8), jnp.tile(ba[H:], 8),
      jnp.tile(bb[:H], 8), jnp.tile(bb[H:], 8),
      jnp.tile(g[:H] * BN_SCALE, 8), jnp.tile(g[H:] * BN_SCALE, 8),
      jnp.tile(b[:H], 8), jnp.tile(b[H:], 8),
  ])


def _layer(y, a, ba, wb, bb, g, b, wn):
  return pl.pallas_call(
      _layer_body,
      grid=(PR // RB,),
      in_specs=[
          pl.BlockSpec((2, RB, 128), lambda i: (0, i, 0)),
          pl.BlockSpec((2, RB, 128), lambda i: (0, i, 0)),
          pl.BlockSpec((4, 128, 128), lambda i: (0, 0, 0)),
          pl.BlockSpec((4, 128, 128), lambda i: (0, 0, 0)),
          pl.BlockSpec((8, 128), lambda i: (0, 0)),
      ],
      out_specs=pl.BlockSpec((2, RB, 128), lambda i: (0, i, 0)),
      out_shape=jax.ShapeDtypeStruct((2, PR, 128), jnp.float32),
  )(y, a, _quads(wb), _quads(wn), _mk_bv(ba, bb, g, b))


def _final(y, a, ba, wb, bb, g, b):
  return pl.pallas_call(
      _final_body,
      grid=(PR // RB,),
      in_specs=[
          pl.BlockSpec((2, RB, 128), lambda i: (0, i, 0)),
          pl.BlockSpec((2, RB, 128), lambda i: (0, i, 0)),
          pl.BlockSpec((4, 128, 128), lambda i: (0, 0, 0)),
          pl.BlockSpec((8, 128), lambda i: (0, 0)),
      ],
      out_specs=pl.BlockSpec((2, RB, 128), lambda i: (0, i, 0)),
      out_shape=jax.ShapeDtypeStruct((2, PR, 128), jnp.float32),
  )(y, a, _quads(wb), _mk_bv(ba, bb, g, b))


def _head_body(p_ref, wu_ref, c_ref, bv_ref, b2_ref, o_ref):
  wu = wu_ref[...]
  bv = bv_ref[...]
  p0 = p_ref[0]
  p1 = p_ref[1]
  h0 = jnp.dot(p0, wu[0], preferred_element_type=jnp.float32)
  h0 += jnp.dot(p1, wu[1], preferred_element_type=jnp.float32)
  h1 = jnp.dot(p0, wu[2], preferred_element_type=jnp.float32)
  h1 += jnp.dot(p1, wu[3], preferred_element_type=jnp.float32)
  h0 = jnp.maximum(h0 + bv[0:1], 0.0)
  h1 = jnp.maximum(h1 + bv[1:2], 0.0)
  o = jnp.dot(h0, c_ref[0], preferred_element_type=jnp.float32)
  o += jnp.dot(h1, c_ref[1], preferred_element_type=jnp.float32)
  o_ref[...] = jnp.tanh(o + b2_ref[...])


def _head(pooled_pk, w1, b1, w2, b2):
  c2 = jnp.stack([_blk(w2[:H, :].reshape(H, 1)).reshape(128, 8),
                  _blk(w2[H:, :].reshape(H, 1)).reshape(128, 8)])
  bv = jnp.stack([jnp.tile(b1[:H], 8), jnp.tile(b1[H:], 8)])
  return pl.pallas_call(
      _head_body,
      out_shape=jax.ShapeDtypeStruct((GP // 8, 8), jnp.float32),
  )(pooled_pk, _quads(w1), c2, bv, b2.reshape(1, 1))


def kernel(x, edge_index, batch,
           c1_Wa, c1_ba, c1_Wb, c1_bb, bn1_g, bn1_b,
           c2_Wa, c2_ba, c2_Wb, c2_bb, bn2_g, bn2_b,
           c3_Wa, c3_ba, c3_Wb, c3_bb, bn3_g, bn3_b,
           fc1_W, fc1_b, fc2_W, fc2_b):
  edges = edge_index.astype(jnp.int32).reshape(2 * E)
  bpad = jnp.concatenate(
      [batch.astype(jnp.int32), jnp.full((NP - N,), G - 1, jnp.int32)]
  ).reshape(NPR, 128)
  xpk = jnp.pad(x, ((0, NP - N), (0, H - 3))).reshape(PR, 128)
  ablk = jnp.zeros((16, D), jnp.float32).at[:3, :].set(c1_Wa)
  a2 = jnp.stack([_blk(ablk[:, :H]), _blk(ablk[:, H:])])

  agg = _agg_call()
  y = _embed(xpk, a2)
  a = agg(y.reshape(NC * NP, H), edges).reshape(2, PR, 128)
  y = _layer(y, a, c1_ba, c1_Wb, c1_bb, bn1_g, bn1_b, c2_Wa)
  a = agg(y.reshape(NC * NP, H), edges).reshape(2, PR, 128)
  y = _layer(y, a, c2_ba, c2_Wb, c2_bb, bn2_g, bn2_b, c3_Wa)
  a = agg(y.reshape(NC * NP, H), edges).reshape(2, PR, 128)
  h3 = _final(y, a, c3_ba, c3_Wb, c3_bb, bn3_g, bn3_b)
  pooled = _pool_call()(h3.reshape(NC * NP, H), bpad).reshape(2, GP // 8, 128)
  out = _head(pooled, fc1_W, fc1_b, fc2_W, fc2_b)
  return out.reshape(GP, 1)[:G]


# speedup vs baseline: 22.5408x; 1.3934x over previous
import functools

import jax
import jax.numpy as jnp
import numpy as np
from jax import lax
from jax.experimental import pallas as pl
from jax.experimental.pallas import tpu as pltpu
from jax.experimental.pallas import tpu_sc as plsc

N = 100000
E = 1600000
G = 1000
GP = 1024
D = 32
H = 16
NP = 102400
NPR = NP // 128
NC = 2
NS = 16
R = 1024
BN_SCALE = float(1.0 / (1.0 + 1e-5) ** 0.5)

CHUNK = 512
EP = 1638400
ETILE = EP // NS
NSUP = ETILE // CHUNK

_PAD_IDX = np.asarray(N + np.arange(EP - E) % 2048, dtype=np.int32)

_sc_mesh = functools.partial(
    plsc.VectorSubcoreMesh,
    core_axis_name="c", subcore_axis_name="s", num_cores=NC, num_subcores=NS,
)
_SC_PARAMS = pltpu.CompilerParams(use_tc_tiling_on_sc=False)


def _agg_body(y_hbm, edges_hbm, out_hbm, acc, sidx, didx, rows, zbuf,
              gsem0, gsem1, ssem0, ssem1, isem0, isem1, isem2, isem3):
  c = lax.axis_index("c")
  s = lax.axis_index("s")
  gsems = (gsem0, gsem1)
  ssems = (ssem0, ssem1)
  isems = (isem0, isem1, isem2, isem3)

  def zfill(i, carry):
    zbuf[i] = jnp.zeros((H,), jnp.float32)
    return carry
  lax.fori_loop(0, 128, zfill, 0)
  zrow0 = s * (NP // NS)
  for k in range(50):
    pltpu.sync_copy(zbuf, acc.at[pl.ds(zrow0 + k * 128, 128)])
  plsc.subcore_barrier()

  coff = c * NP
  e0 = s * ETILE

  def idx_fire(u, m):
    base = e0 + u * CHUNK
    pltpu.async_copy(edges_hbm.at[pl.ds(base, CHUNK)], sidx.at[m], isems[m])
    pltpu.async_copy(edges_hbm.at[pl.ds(EP + base, CHUNK)], didx.at[m],
                     isems[m])

  def idx_wait_gather_fire(u, m, k):
    base = e0 + u * CHUNK
    pltpu.make_async_copy(
        edges_hbm.at[pl.ds(base, CHUNK)], sidx.at[m], isems[m]).wait()
    pltpu.make_async_copy(
        edges_hbm.at[pl.ds(EP + base, CHUNK)], didx.at[m], isems[m]).wait()
    for q in range(CHUNK // 16):
      sidx[m, pl.ds(q * 16, 16)] = sidx[m, pl.ds(q * 16, 16)] + coff
    pltpu.async_copy(y_hbm.at[sidx.at[m]], rows.at[k], gsems[k])

  def gather_wait_scatter_fire(m, k):
    pltpu.make_async_copy(y_hbm.at[sidx.at[m]], rows.at[k], gsems[k]).wait()
    pltpu.async_copy(rows.at[k], acc.at[didx.at[m]], ssems[k], add=True)

  def scatter_wait(m, k):
    pltpu.make_async_copy(rows.at[k], acc.at[didx.at[m]], ssems[k]).wait()

  for m in range(4):
    idx_fire(m, m)
  idx_wait_gather_fire(0, 0, 0)
  idx_wait_gather_fire(1, 1, 1)

  def pipe(h, carry):
    a = 4 * h
    gather_wait_scatter_fire(0, 0)
    gather_wait_scatter_fire(1, 1)
    scatter_wait(0, 0)
    idx_wait_gather_fire(a + 2, 2, 0)
    idx_fire(a + 4, 0)
    scatter_wait(1, 1)
    idx_wait_gather_fire(a + 3, 3, 1)
    idx_fire(a + 5, 1)
    gather_wait_scatter_fire(2, 0)
    gather_wait_scatter_fire(3, 1)
    scatter_wait(2, 0)
    idx_wait_gather_fire(a + 4, 0, 0)
    idx_fire(a + 6, 2)
    scatter_wait(3, 1)
    idx_wait_gather_fire(a + 5, 1, 1)
    idx_fire(a + 7, 3)
    return carry

  lax.fori_loop(0, NSUP // 4 - 1, pipe, 0)
  a = NSUP - 4
  gather_wait_scatter_fire(0, 0)
  gather_wait_scatter_fire(1, 1)
  scatter_wait(0, 0)
  idx_wait_gather_fire(a + 2, 2, 0)
  scatter_wait(1, 1)
  idx_wait_gather_fire(a + 3, 3, 1)
  gather_wait_scatter_fire(2, 0)
  gather_wait_scatter_fire(3, 1)
  scatter_wait(2, 0)
  scatter_wait(3, 1)
  plsc.subcore_barrier()
  wrow0 = s * (NP // NS)
  pltpu.sync_copy(acc.at[pl.ds(wrow0, NP // NS)],
                  out_hbm.at[pl.ds(coff + wrow0, NP // NS)])


@functools.cache
def _agg_call():
  return pl.kernel(
      _agg_body,
      out_type=jax.ShapeDtypeStruct((NC * NP, H), jnp.float32),
      mesh=_sc_mesh(),
      scratch_types=[
          pltpu.VMEM_SHARED((NP, H), jnp.float32),
          pltpu.VMEM((4, CHUNK), jnp.int32),
          pltpu.VMEM((4, CHUNK), jnp.int32),
          pltpu.VMEM((2, CHUNK, H), jnp.float32),
          pltpu.VMEM((128, H), jnp.float32),
      ] + [pltpu.SemaphoreType.DMA] * 8,
      compiler_params=_SC_PARAMS,
  )


def _pool_body(h_hbm, b_hbm, out_hbm, acc, bidx, vrows, zbuf, gsem):
  c = lax.axis_index("c")
  s = lax.axis_index("s")

  def zfill(i, carry):
    zbuf[i] = jnp.zeros((H,), jnp.float32)
    return carry
  lax.fori_loop(0, 64, zfill, 0)
  pltpu.sync_copy(zbuf, acc.at[pl.ds(s * 64, 64)])
  plsc.subcore_barrier()

  def stage(u, carry):
    g = s + u * NS
    pltpu.sync_copy(b_hbm.at[pl.ds(g * 8, 8)], bidx)
    pltpu.sync_copy(h_hbm.at[pl.ds(c * NP + g * 1024, 1024)], vrows)
    for r in range(8):
      pltpu.sync_copy(vrows.at[pl.ds(r * 128, 128)], acc.at[bidx.at[r]],
                      add=True)
    return carry
  lax.fori_loop(0, 6 + (s < 4).astype(jnp.int32), stage, 0)

  plsc.subcore_barrier()
  pltpu.sync_copy(acc.at[pl.ds(s * 64, 64)],
                  out_hbm.at[pl.ds(c * GP + s * 64, 64)])


@functools.cache
def _pool_call():
  return pl.kernel(
      _pool_body,
      out_type=jax.ShapeDtypeStruct((NC * GP, H), jnp.float32),
      mesh=_sc_mesh(),
      scratch_types=[
          pltpu.VMEM_SHARED((GP, H), jnp.float32),
          pltpu.VMEM((8, 128), jnp.int32),
          pltpu.VMEM((1024, H), jnp.float32),
          pltpu.VMEM((64, H), jnp.float32),
          pltpu.SemaphoreType.DMA,
      ],
      compiler_params=_SC_PARAMS,
  )


PR = NP // 8
RB = 1280


def _blk(m16):
  return jnp.kron(jnp.eye(8, dtype=jnp.float32), m16)


def _quads(w):
  return jnp.stack([_blk(w[:H, :H]), _blk(w[H:, :H]),
                    _blk(w[:H, H:]), _blk(w[H:, H:])])


def _tile8(v16):
  return jnp.tile(v16, 8).reshape(1, 128)


def _embed_body(x_ref, a_ref, o_ref):
  xb = x_ref[...]
  o_ref[0] = jnp.dot(xb, a_ref[0], preferred_element_type=jnp.float32)
  o_ref[1] = jnp.dot(xb, a_ref[1], preferred_element_type=jnp.float32)


def _embed(xpk, a2):
  return pl.pallas_call(
      _embed_body,
      grid=(PR // RB,),
      in_specs=[
          pl.BlockSpec((RB, 128), lambda i: (i, 0)),
          pl.BlockSpec((2, 128, 128), lambda i: (0, 0, 0)),
      ],
      out_specs=pl.BlockSpec((2, RB, 128), lambda i: (0, i, 0)),
      out_shape=jax.ShapeDtypeStruct((2, PR, 128), jnp.float32),
  )(xpk, a2)


def _mlp_packed(y_ref, a_ref, wu, bv):
  h0 = jnp.maximum(y_ref[0] + a_ref[0] + bv[0:1], 0.0)
  h1 = jnp.maximum(y_ref[1] + a_ref[1] + bv[1:2], 0.0)
  u0 = jnp.dot(h0, wu[0], preferred_element_type=jnp.float32)
  u0 += jnp.dot(h1, wu[1], preferred_element_type=jnp.float32)
  u1 = jnp.dot(h0, wu[2], preferred_element_type=jnp.float32)
  u1 += jnp.dot(h1, wu[3], preferred_element_type=jnp.float32)
  u0 = jnp.maximum(u0 + bv[2:3], 0.0)
  u1 = jnp.maximum(u1 + bv[3:4], 0.0)
  v0 = u0 * bv[4:5] + bv[6:7]
  v1 = u1 * bv[5:6] + bv[7:8]
  return v0, v1


def _layer_body(y_ref, a_ref, wu_ref, wz_ref, bv_ref, o_ref):
  wu = wu_ref[...]
  wz = wz_ref[...]
  v0, v1 = _mlp_packed(y_ref, a_ref, wu, bv_ref[...])
  z0 = jnp.dot(v0, wz[0], preferred_element_type=jnp.float32)
  z0 += jnp.dot(v1, wz[1], preferred_element_type=jnp.float32)
  z1 = jnp.dot(v0, wz[2], preferred_element_type=jnp.float32)
  z1 += jnp.dot(v1, wz[3], preferred_element_type=jnp.float32)
  o_ref[0] = z0
  o_ref[1] = z1


def _final_body(y_ref, a_ref, wu_ref, bv_ref, o_ref):
  i = pl.program_id(0)
  v0, v1 = _mlp_packed(y_ref, a_ref, wu_ref[...], bv_ref[...])
  row = i * RB + lax.broadcasted_iota(jnp.int32, (RB, 128), 0)
  node = row * 8 + lax.broadcasted_iota(jnp.int32, (RB, 128), 1) // H
  keep = node < N
  o_ref[0] = jnp.where(keep, v0, 0.0)
  o_ref[1] = jnp.where(keep, v1, 0.0)


def _mk_bv(ba, bb, g, b):
  return jnp.stack([
      jnp.tile(ba[:H], 8), jnp.tile(ba[H:], 8),
      jnp.tile(bb[:H], 8), jnp.tile(bb[H:], 8),
      jnp.tile(g[:H] * BN_SCALE, 8), jnp.tile(g[H:] * BN_SCALE, 8),
      jnp.tile(b[:H], 8), jnp.tile(b[H:], 8),
  ])


def _layer(y, a, ba, wb, bb, g, b, wn):
  return pl.pallas_call(
      _layer_body,
      grid=(PR // RB,),
      in_specs=[
          pl.BlockSpec((2, RB, 128), lambda i: (0, i, 0)),
          pl.BlockSpec((2, RB, 128), lambda i: (0, i, 0)),
          pl.BlockSpec((4, 128, 128), lambda i: (0, 0, 0)),
          pl.BlockSpec((4, 128, 128), lambda i: (0, 0, 0)),
          pl.BlockSpec((8, 128), lambda i: (0, 0)),
      ],
      out_specs=pl.BlockSpec((2, RB, 128), lambda i: (0, i, 0)),
      out_shape=jax.ShapeDtypeStruct((2, PR, 128), jnp.float32),
  )(y, a, _quads(wb), _quads(wn), _mk_bv(ba, bb, g, b))


def _final(y, a, ba, wb, bb, g, b):
  return pl.pallas_call(
      _final_body,
      grid=(PR // RB,),
      in_specs=[
          pl.BlockSpec((2, RB, 128), lambda i: (0, i, 0)),
          pl.BlockSpec((2, RB, 128), lambda i: (0, i, 0)),
          pl.BlockSpec((4, 128, 128), lambda i: (0, 0, 0)),
          pl.BlockSpec((8, 128), lambda i: (0, 0)),
      ],
      out_specs=pl.BlockSpec((2, RB, 128), lambda i: (0, i, 0)),
      out_shape=jax.ShapeDtypeStruct((2, PR, 128), jnp.float32),
  )(y, a, _quads(wb), _mk_bv(ba, bb, g, b))


def _head_body(p_ref, wu_ref, c_ref, bv_ref, b2_ref, o_ref):
  wu = wu_ref[...]
  bv = bv_ref[...]
  p0 = p_ref[0]
  p1 = p_ref[1]
  h0 = jnp.dot(p0, wu[0], preferred_element_type=jnp.float32)
  h0 += jnp.dot(p1, wu[1], preferred_element_type=jnp.float32)
  h1 = jnp.dot(p0, wu[2], preferred_element_type=jnp.float32)
  h1 += jnp.dot(p1, wu[3], preferred_element_type=jnp.float32)
  h0 = jnp.maximum(h0 + bv[0:1], 0.0)
  h1 = jnp.maximum(h1 + bv[1:2], 0.0)
  o = jnp.dot(h0, c_ref[0], preferred_element_type=jnp.float32)
  o += jnp.dot(h1, c_ref[1], preferred_element_type=jnp.float32)
  o_ref[...] = jnp.tanh(o + b2_ref[...])


def _head(pooled_pk, w1, b1, w2, b2):
  c2 = jnp.stack([_blk(w2[:H, :].reshape(H, 1)).reshape(128, 8),
                  _blk(w2[H:, :].reshape(H, 1)).reshape(128, 8)])
  bv = jnp.stack([jnp.tile(b1[:H], 8), jnp.tile(b1[H:], 8)])
  return pl.pallas_call(
      _head_body,
      out_shape=jax.ShapeDtypeStruct((GP // 8, 8), jnp.float32),
  )(pooled_pk, _quads(w1), c2, bv, b2.reshape(1, 1))


def kernel(x, edge_index, batch,
           c1_Wa, c1_ba, c1_Wb, c1_bb, bn1_g, bn1_b,
           c2_Wa, c2_ba, c2_Wb, c2_bb, bn2_g, bn2_b,
           c3_Wa, c3_ba, c3_Wb, c3_bb, bn3_g, bn3_b,
           fc1_W, fc1_b, fc2_W, fc2_b):
  pad_idx = jnp.asarray(_PAD_IDX)
  ei = edge_index.astype(jnp.int32)
  edges = jnp.concatenate([ei[0], pad_idx, ei[1], pad_idx])
  bpad = jnp.concatenate(
      [batch.astype(jnp.int32), jnp.full((NP - N,), G - 1, jnp.int32)]
  ).reshape(NPR, 128)
  xpk = jnp.pad(x, ((0, NP - N), (0, H - 3))).reshape(PR, 128)
  ablk = jnp.zeros((16, D), jnp.float32).at[:3, :].set(c1_Wa)
  a2 = jnp.stack([_blk(ablk[:, :H]), _blk(ablk[:, H:])])

  agg = _agg_call()
  y = _embed(xpk, a2)
  a = agg(y.reshape(NC * NP, H), edges).reshape(2, PR, 128)
  y = _layer(y, a, c1_ba, c1_Wb, c1_bb, bn1_g, bn1_b, c2_Wa)
  a = agg(y.reshape(NC * NP, H), edges).reshape(2, PR, 128)
  y = _layer(y, a, c2_ba, c2_Wb, c2_bb, bn2_g, bn2_b, c3_Wa)
  a = agg(y.reshape(NC * NP, H), edges).reshape(2, PR, 128)
  h3 = _final(y, a, c3_ba, c3_Wb, c3_bb, bn3_g, bn3_b)
  pooled = _pool_call()(h3.reshape(NC * NP, H), bpad).reshape(2, GP // 8, 128)
  out = _head(pooled, fc1_W, fc1_b, fc2_W, fc2_b)
  return out.reshape(GP, 1)[:G]
